# baseline (device time: 103157 ns/iter reference)
import jax
import jax.numpy as jnp
from jax import lax
from jax.experimental import pallas as pl
from jax.experimental.pallas import tpu as pltpu

N_DEV = 16
H = 8
P = 4
OUT_SLOTS = 4


def _carried(h, p, cw):
    if h < H - 1:
        return True
    return p < P // 2 if cw else p >= P // 2


def kernel(x, w_mat, scale_x, scale_w):
    m_per, k = x.shape
    k_w, n_per = w_mat.shape
    m_piece = m_per // P

    def body(x_ref, w_ref, sx_ref, sw_ref, out_ref, xg_ref, w_stage,
             w8_ref, out_stage, cw_send, cw_recv, ccw_send, ccw_recv,
             w_sem, out_sems):
        my = lax.axis_index("i")
        left = lax.rem(my + N_DEV - 1, N_DEV)
        right = lax.rem(my + 1, N_DEV)

        def slot(off):
            return lax.rem(my + N_DEV + off, N_DEV)

        def fwd(s, h, p, cw):
            return pltpu.make_async_remote_copy(
                src_ref=xg_ref.at[s, p],
                dst_ref=xg_ref.at[s, p],
                send_sem=(cw_send if cw else ccw_send).at[h, p],
                recv_sem=(cw_recv if cw else ccw_recv).at[h, p],
                device_id=(right if cw else left,),
                device_id_type=pl.DeviceIdType.MESH,
            )

        half = k_w // 2
        w_dma0 = pltpu.make_async_copy(
            w_ref.at[pl.ds(0, half), :], w_stage, w_sem)
        w_dma0.start()

        barrier_sem = pltpu.get_barrier_semaphore()
        for nbr in (left, right):
            pl.semaphore_signal(
                barrier_sem, inc=1,
                device_id=(nbr,), device_id_type=pl.DeviceIdType.MESH,
            )
        pl.semaphore_wait(barrier_sem, 2)

        xg_ref[pl.ds(my, 1)] = (
            x_ref[...].astype(jnp.float8_e4m3fn)
            .reshape(1, P, m_piece, k)
        )
        pending = []

        def start(s, h, p, cw):
            d = fwd(s, h, p, cw)
            d.start()
            pending.append(d)

        for p in range(P):
            start(slot(0), 0, p, cw=True)
            start(slot(0), 0, p, cw=False)

        w_dma0.wait()
        w8_ref[pl.ds(0, half), :] = w_stage[...].astype(jnp.float8_e5m2)
        w_dma1 = pltpu.make_async_copy(
            w_ref.at[pl.ds(half, half), :], w_stage, w_sem)
        w_dma1.start()
        w_dma1.wait()
        w8_ref[pl.ds(half, half), :] = w_stage[...].astype(jnp.float8_e5m2)

        scale = sx_ref[0] * sw_ref[0]
        out_dmas = []

        def compute_slab(s):
            j = len(out_dmas)
            sl = j % OUT_SLOTS
            if j >= OUT_SLOTS:
                out_dmas[j - OUT_SLOTS].wait()
            acc = jnp.dot(
                xg_ref[s].reshape(m_per, k), w8_ref[...],
                preferred_element_type=jnp.float32,
            )
            out_stage[pl.ds(sl, 1)] = (acc * scale)[None]
            d = pltpu.make_async_copy(
                out_stage.at[sl],
                out_ref.at[pl.ds(s * m_per, m_per), :],
                out_sems.at[sl],
            )
            d.start()
            out_dmas.append(d)

        compute_slab(slot(0))

        for h in range(H):
            for p in range(P):
                for cw in (True, False):
                    if not _carried(h, p, cw):
                        continue
                    s = slot(-h - 1) if cw else slot(h + 1)
                    recv = fwd(s, h, p, cw)
                    recv.wait_recv()
                    if h + 1 < H and _carried(h + 1, p, cw):
                        start(s, h + 1, p, cw)
            if h < H - 1:
                compute_slab(slot(-h - 1))
                compute_slab(slot(h + 1))
            else:
                compute_slab(slot(H))

        for d in pending:
            d.wait_send()
        for d in out_dmas[-OUT_SLOTS:]:
            d.wait()

    return pl.pallas_call(
        body,
        out_shape=jax.ShapeDtypeStruct((N_DEV * m_per, n_per), jnp.float32),
        in_specs=[
            pl.BlockSpec(memory_space=pltpu.VMEM),
            pl.BlockSpec(memory_space=pltpu.MemorySpace.HBM),
            pl.BlockSpec(memory_space=pltpu.SMEM),
            pl.BlockSpec(memory_space=pltpu.SMEM),
        ],
        out_specs=pl.BlockSpec(memory_space=pltpu.MemorySpace.HBM),
        scratch_shapes=[
            pltpu.VMEM((N_DEV, P, m_piece, k), jnp.float8_e4m3fn),
            pltpu.VMEM((k_w // 2, n_per), jnp.float32),
            pltpu.VMEM((k_w, n_per), jnp.float8_e5m2),
            pltpu.VMEM((OUT_SLOTS, m_per, n_per), jnp.float32),
            pltpu.SemaphoreType.DMA((H, P)),
            pltpu.SemaphoreType.DMA((H, P)),
            pltpu.SemaphoreType.DMA((H, P)),
            pltpu.SemaphoreType.DMA((H, P)),
            pltpu.SemaphoreType.DMA,
            pltpu.SemaphoreType.DMA((OUT_SLOTS,)),
        ],
        compiler_params=pltpu.CompilerParams(collective_id=0),
    )(x, w_mat, scale_x, scale_w)


# device time: 98574 ns/iter; 1.0465x vs baseline; 1.0465x over previous
import jax
import jax.numpy as jnp
from jax import lax
from jax.experimental import pallas as pl
from jax.experimental.pallas import tpu as pltpu

N_DEV = 16
H = 8
P = 4
OUT_SLOTS = 4


def _carried(h, p, cw):
    if h < H - 1:
        return True
    return p < P // 2 if cw else p >= P // 2


def kernel(x, w_mat, scale_x, scale_w):
    m_per, k = x.shape
    k_w, n_per = w_mat.shape
    m_piece = m_per // P

    def body(x_ref, w_ref, sx_ref, sw_ref, out_ref, xg_ref, x_stage, w_stage,
             w8_ref, out_stage, cw_send, cw_recv, ccw_send, ccw_recv,
             x_sem, w_sem, out_sems):
        my = lax.axis_index("i")
        left = lax.rem(my + N_DEV - 1, N_DEV)
        right = lax.rem(my + 1, N_DEV)

        x_dma = pltpu.make_async_copy(x_ref, x_stage, x_sem)
        x_dma.start()

        def slot(off):
            return lax.rem(my + N_DEV + off, N_DEV)

        def fwd(s, h, p, cw):
            return pltpu.make_async_remote_copy(
                src_ref=xg_ref.at[s, p],
                dst_ref=xg_ref.at[s, p],
                send_sem=(cw_send if cw else ccw_send).at[h, p],
                recv_sem=(cw_recv if cw else ccw_recv).at[h, p],
                device_id=(right if cw else left,),
                device_id_type=pl.DeviceIdType.MESH,
            )

        half = k_w // 2
        w_dma0 = pltpu.make_async_copy(
            w_ref.at[pl.ds(0, half), :], w_stage, w_sem)
        w_dma0.start()

        barrier_sem = pltpu.get_barrier_semaphore()
        for nbr in (left, right):
            pl.semaphore_signal(
                barrier_sem, inc=1,
                device_id=(nbr,), device_id_type=pl.DeviceIdType.MESH,
            )
        pl.semaphore_wait(barrier_sem, 2)

        x_dma.wait()
        xg_ref[pl.ds(my, 1)] = (
            x_stage[...].astype(jnp.float8_e4m3fn)
            .reshape(1, P, m_piece, k)
        )
        pending = []

        def start(s, h, p, cw):
            d = fwd(s, h, p, cw)
            d.start()
            pending.append(d)

        for p in range(P):
            start(slot(0), 0, p, cw=True)
            start(slot(0), 0, p, cw=False)

        w_dma0.wait()
        w8_ref[pl.ds(0, half), :] = w_stage[...].astype(jnp.float8_e5m2)
        w_dma1 = pltpu.make_async_copy(
            w_ref.at[pl.ds(half, half), :], w_stage, w_sem)
        w_dma1.start()
        w_dma1.wait()
        w8_ref[pl.ds(half, half), :] = w_stage[...].astype(jnp.float8_e5m2)

        scale = sx_ref[0] * sw_ref[0]
        out_dmas = []

        def compute_slab(s):
            j = len(out_dmas)
            sl = j % OUT_SLOTS
            if j >= OUT_SLOTS:
                out_dmas[j - OUT_SLOTS].wait()
            acc = jnp.dot(
                xg_ref[s].reshape(m_per, k), w8_ref[...],
                preferred_element_type=jnp.float32,
            )
            out_stage[pl.ds(sl, 1)] = (acc * scale)[None]
            d = pltpu.make_async_copy(
                out_stage.at[sl],
                out_ref.at[pl.ds(s * m_per, m_per), :],
                out_sems.at[sl],
            )
            d.start()
            out_dmas.append(d)

        compute_slab(slot(0))

        for h in range(H):
            for p in range(P):
                for cw in (True, False):
                    if not _carried(h, p, cw):
                        continue
                    s = slot(-h - 1) if cw else slot(h + 1)
                    recv = fwd(s, h, p, cw)
                    recv.wait_recv()
                    if h + 1 < H and _carried(h + 1, p, cw):
                        start(s, h + 1, p, cw)
            if h < H - 1:
                compute_slab(slot(-h - 1))
                compute_slab(slot(h + 1))
            else:
                compute_slab(slot(H))

        for d in pending:
            d.wait_send()
        for d in out_dmas[-OUT_SLOTS:]:
            d.wait()

    return pl.pallas_call(
        body,
        out_shape=jax.ShapeDtypeStruct((N_DEV * m_per, n_per), jnp.float32),
        in_specs=[
            pl.BlockSpec(memory_space=pltpu.MemorySpace.HBM),
            pl.BlockSpec(memory_space=pltpu.MemorySpace.HBM),
            pl.BlockSpec(memory_space=pltpu.SMEM),
            pl.BlockSpec(memory_space=pltpu.SMEM),
        ],
        out_specs=pl.BlockSpec(memory_space=pltpu.MemorySpace.HBM),
        scratch_shapes=[
            pltpu.VMEM((N_DEV, P, m_piece, k), jnp.float8_e4m3fn),
            pltpu.VMEM((m_per, k), jnp.float32),
            pltpu.VMEM((k_w // 2, n_per), jnp.float32),
            pltpu.VMEM((k_w, n_per), jnp.float8_e5m2),
            pltpu.VMEM((OUT_SLOTS, m_per, n_per), jnp.float32),
            pltpu.SemaphoreType.DMA((H, P)),
            pltpu.SemaphoreType.DMA((H, P)),
            pltpu.SemaphoreType.DMA((H, P)),
            pltpu.SemaphoreType.DMA((H, P)),
            pltpu.SemaphoreType.DMA,
            pltpu.SemaphoreType.DMA,
            pltpu.SemaphoreType.DMA((OUT_SLOTS,)),
        ],
        compiler_params=pltpu.CompilerParams(collective_id=0),
    )(
        pltpu.with_memory_space_constraint(x, pltpu.MemorySpace.HBM),
        pltpu.with_memory_space_constraint(w_mat, pltpu.MemorySpace.HBM),
        scale_x,
        scale_w,
    )


# device time: 97473 ns/iter; 1.0583x vs baseline; 1.0113x over previous
import jax
import jax.numpy as jnp
from jax import lax
from jax.experimental import pallas as pl
from jax.experimental.pallas import tpu as pltpu

N_DEV = 16
H = 8
P = 4
OUT_SLOTS = 4


def _carried(h, p, cw):
    if h < H - 1:
        return True
    return p < P // 2 if cw else p >= P // 2


def kernel(x, w_mat, scale_x, scale_w):
    m_per, k = x.shape
    k_w, n_per = w_mat.shape
    m_piece = m_per // P

    def body(x_ref, w_ref, sx_ref, sw_ref, out_ref, xg_ref, x_stage, w_stage,
             w8_ref, out_stage, cw_send, cw_recv, ccw_send, ccw_recv,
             x_sem, w_sem, out_sems):
        my = lax.axis_index("i")
        left = lax.rem(my + N_DEV - 1, N_DEV)
        right = lax.rem(my + 1, N_DEV)

        x_dmas = []
        for p in range(P):
            d = pltpu.make_async_copy(
                x_ref.at[pl.ds(p * m_piece, m_piece), :],
                x_stage.at[p],
                x_sem.at[p],
            )
            d.start()
            x_dmas.append(d)

        def slot(off):
            return lax.rem(my + N_DEV + off, N_DEV)

        def fwd(s, h, p, cw):
            return pltpu.make_async_remote_copy(
                src_ref=xg_ref.at[s, p],
                dst_ref=xg_ref.at[s, p],
                send_sem=(cw_send if cw else ccw_send).at[h, p],
                recv_sem=(cw_recv if cw else ccw_recv).at[h, p],
                device_id=(right if cw else left,),
                device_id_type=pl.DeviceIdType.MESH,
            )

        half = k_w // 2
        w_dma0 = pltpu.make_async_copy(
            w_ref.at[pl.ds(0, half), :], w_stage, w_sem)
        w_dma0.start()

        barrier_sem = pltpu.get_barrier_semaphore()
        for nbr in (left, right):
            pl.semaphore_signal(
                barrier_sem, inc=1,
                device_id=(nbr,), device_id_type=pl.DeviceIdType.MESH,
            )
        pl.semaphore_wait(barrier_sem, 2)

        pending = []

        def start(s, h, p, cw):
            d = fwd(s, h, p, cw)
            d.start()
            pending.append(d)

        for p in range(P):
            x_dmas[p].wait()
            xg_ref[pl.ds(my, 1), pl.ds(p, 1)] = (
                x_stage[p].astype(jnp.float8_e4m3fn)[None, None]
            )
            start(slot(0), 0, p, cw=True)
            start(slot(0), 0, p, cw=False)

        w_dma0.wait()
        w8_ref[pl.ds(0, half), :] = w_stage[...].astype(jnp.float8_e5m2)
        w_dma1 = pltpu.make_async_copy(
            w_ref.at[pl.ds(half, half), :], w_stage, w_sem)
        w_dma1.start()
        w_dma1.wait()
        w8_ref[pl.ds(half, half), :] = w_stage[...].astype(jnp.float8_e5m2)

        scale = sx_ref[0] * sw_ref[0]
        out_dmas = []

        def compute_slab(s):
            j = len(out_dmas)
            sl = j % OUT_SLOTS
            if j >= OUT_SLOTS:
                out_dmas[j - OUT_SLOTS].wait()
            acc = jnp.dot(
                xg_ref[s].reshape(m_per, k), w8_ref[...],
                preferred_element_type=jnp.float32,
            )
            out_stage[pl.ds(sl, 1)] = (acc * scale)[None]
            d = pltpu.make_async_copy(
                out_stage.at[sl],
                out_ref.at[pl.ds(s * m_per, m_per), :],
                out_sems.at[sl],
            )
            d.start()
            out_dmas.append(d)

        compute_slab(slot(0))

        for h in range(H):
            for p in range(P):
                for cw in (True, False):
                    if not _carried(h, p, cw):
                        continue
                    s = slot(-h - 1) if cw else slot(h + 1)
                    recv = fwd(s, h, p, cw)
                    recv.wait_recv()
                    if h + 1 < H and _carried(h + 1, p, cw):
                        start(s, h + 1, p, cw)
            if h < H - 1:
                compute_slab(slot(-h - 1))
                compute_slab(slot(h + 1))
            else:
                compute_slab(slot(H))

        for d in pending:
            d.wait_send()
        for d in out_dmas[-OUT_SLOTS:]:
            d.wait()

    return pl.pallas_call(
        body,
        out_shape=jax.ShapeDtypeStruct((N_DEV * m_per, n_per), jnp.float32),
        in_specs=[
            pl.BlockSpec(memory_space=pltpu.MemorySpace.HBM),
            pl.BlockSpec(memory_space=pltpu.MemorySpace.HBM),
            pl.BlockSpec(memory_space=pltpu.SMEM),
            pl.BlockSpec(memory_space=pltpu.SMEM),
        ],
        out_specs=pl.BlockSpec(memory_space=pltpu.MemorySpace.HBM),
        scratch_shapes=[
            pltpu.VMEM((N_DEV, P, m_piece, k), jnp.float8_e4m3fn),
            pltpu.VMEM((P, m_piece, k), jnp.float32),
            pltpu.VMEM((k_w // 2, n_per), jnp.float32),
            pltpu.VMEM((k_w, n_per), jnp.float8_e5m2),
            pltpu.VMEM((OUT_SLOTS, m_per, n_per), jnp.float32),
            pltpu.SemaphoreType.DMA((H, P)),
            pltpu.SemaphoreType.DMA((H, P)),
            pltpu.SemaphoreType.DMA((H, P)),
            pltpu.SemaphoreType.DMA((H, P)),
            pltpu.SemaphoreType.DMA((P,)),
            pltpu.SemaphoreType.DMA,
            pltpu.SemaphoreType.DMA((OUT_SLOTS,)),
        ],
        compiler_params=pltpu.CompilerParams(collective_id=0),
    )(
        pltpu.with_memory_space_constraint(x, pltpu.MemorySpace.HBM),
        pltpu.with_memory_space_constraint(w_mat, pltpu.MemorySpace.HBM),
        scale_x,
        scale_w,
    )
